# baseline (speedup 1.0000x reference)
import functools

import jax
import jax.numpy as jnp
import numpy as np
from jax.experimental import pallas as pl
from jax.experimental.pallas import tpu as pltpu
from jax.experimental.pallas import tpu_sc as plsc

B = 8; S = 512; H = 128; INTENT = 128; SLOT = 512
GHD = 16; GOD = 128; NHEAD = 4; TOPK = 3; WINDOW = 2; ALPHA = 0.2
N = S + INTENT + SLOT
NEG = -9e15
F32 = jnp.float32

NW = 32
RW = (B * S) // NW
WPB = S // RW


def _lrelu(x):
    return jnp.where(x >= 0, x, ALPHA * x)


def _elu(x):
    return jnp.where(x > 0, x, jnp.exp(jnp.minimum(x, 0.0)) - 1.0)


def _mm(a, b):
    return jnp.dot(a, b, preferred_element_type=F32)


def _sc_topk(intent_pro, slot_pro):
    mesh = plsc.VectorSubcoreMesh(core_axis_name="c", subcore_axis_name="s")

    @functools.partial(
        pl.kernel,
        mesh=mesh,
        out_type=(
            jax.ShapeDtypeStruct((B, S, 48), F32),
            jax.ShapeDtypeStruct((B, S, 48), jnp.int32),
            jax.ShapeDtypeStruct((B, S, 48), F32),
            jax.ShapeDtypeStruct((B, S, 48), jnp.int32),
        ),
        scratch_types=[
            pltpu.VMEM((32, INTENT), F32),
            pltpu.VMEM((32, SLOT), F32),
            pltpu.VMEM((32, 48), F32),
            pltpu.VMEM((32, 48), jnp.int32),
            pltpu.VMEM((32, 48), F32),
            pltpu.VMEM((32, 48), jnp.int32),
        ],
    )
    def sc_kernel(ip_hbm, sp_hbm, iv_hbm, ii_hbm, sv_hbm, si_hbm,
                  ip_v, sp_v, iv_v, ii_v, sv_v, si_v):
        cid = jax.lax.axis_index("c")
        sid = jax.lax.axis_index("s")
        wid = sid * 2 + cid
        b = wid // WPB
        r0 = (wid % WPB) * RW
        lane = jax.lax.broadcasted_iota(jnp.int32, (16,), 0)
        negf = jnp.float32(-3.0e38)

        def row_top3(ref, r, nchunk, vout, iout):
            z = jnp.zeros((16,), jnp.int32)
            nf = jnp.full((16,), negf, F32)
            b1, b2, b3, i1, i2, i3 = nf, nf, nf, z, z, z
            for ch in range(nchunk):
                v = ref[r, pl.ds(ch * 16, 16)]
                idx = lane + (ch * 16)
                gt1 = v > b1
                gt2 = v > b2
                gt3 = v > b3
                nb1 = jnp.where(gt1, v, b1)
                ni1 = jnp.where(gt1, idx, i1)
                nb2 = jnp.where(gt1, b1, jnp.where(gt2, v, b2))
                ni2 = jnp.where(gt1, i1, jnp.where(gt2, idx, i2))
                nb3 = jnp.where(gt2, b2, jnp.where(gt3, v, b3))
                ni3 = jnp.where(gt2, i2, jnp.where(gt3, idx, i3))
                b1, i1, b2, i2, b3, i3 = nb1, ni1, nb2, ni2, nb3, ni3
            vout[r, pl.ds(0, 16)] = b1
            vout[r, pl.ds(16, 16)] = b2
            vout[r, pl.ds(32, 16)] = b3
            iout[r, pl.ds(0, 16)] = i1
            iout[r, pl.ds(16, 16)] = i2
            iout[r, pl.ds(32, 16)] = i3

        def row_body(r, carry):
            row_top3(ip_v, r, INTENT // 16, iv_v, ii_v)
            row_top3(sp_v, r, SLOT // 16, sv_v, si_v)
            return carry

        for blk in range(RW // 32):
            rb = r0 + blk * 32
            pltpu.sync_copy(ip_hbm.at[b, pl.ds(rb, 32)], ip_v)
            pltpu.sync_copy(sp_hbm.at[b, pl.ds(rb, 32)], sp_v)
            jax.lax.fori_loop(0, 32, row_body, 0)
            pltpu.sync_copy(iv_v, iv_hbm.at[b, pl.ds(rb, 32)])
            pltpu.sync_copy(ii_v, ii_hbm.at[b, pl.ds(rb, 32)])
            pltpu.sync_copy(sv_v, sv_hbm.at[b, pl.ds(rb, 32)])
            pltpu.sync_copy(si_v, si_hbm.at[b, pl.ds(rb, 32)])

    return sc_kernel(intent_pro, slot_pro)


def _masked_softmax_mm(mask, epre, ub, h_aug, width):
    p = jnp.where(mask, jnp.exp(_lrelu(epre) - ub), 0.0)
    pm = _mm(p, h_aug)
    return pm[:, :width] / pm[:, width:width + 1]


def _merge48(vals, idxs, width):
    cols = []
    for _ in range(TOPK):
        m = jnp.max(vals, axis=1, keepdims=True)
        cand = jnp.where(vals >= m, idxs, width)
        g = jnp.min(cand, axis=1, keepdims=True)
        cols.append(g)
        vals = jnp.where(idxs == g, -3.0e38, vals)
    return cols


def _main_body(iv_ref, ii_ref, sv_ref, si_ref, hid_ref, iemb_ref, semb_ref,
               wcat_ref, a1m_ref, a2m_ref,
               wout_ref, aoutc_ref, iw_ref, ib_ref, sw_ref, sb_ref,
               hidden_ref, iout_ref, sout_ref):
    ii_cols = _merge48(iv_ref[0], ii_ref[0], INTENT)
    si_cols = _merge48(sv_ref[0], si_ref[0], SLOT)

    iotaI = jax.lax.broadcasted_iota(jnp.int32, (S, INTENT), 1)
    iotaS = jax.lax.broadcasted_iota(jnp.int32, (S, SLOT), 1)
    int_oh = (iotaI == ii_cols[0]) | (iotaI == ii_cols[1]) | (iotaI == ii_cols[2])
    slot_oh = (iotaS == si_cols[0]) | (iotaS == si_cols[1]) | (iotaS == si_cols[2])
    int_oh_T = int_oh.astype(F32)
    slot_oh_T = slot_oh.astype(F32)

    rr = jax.lax.broadcasted_iota(jnp.int32, (S, N), 0)
    cc = jax.lax.broadcasted_iota(jnp.int32, (S, N), 1)
    m_seq = (cc >= rr - WINDOW) & (cc <= rr + WINDOW)
    for k in range(TOPK):
        m_seq = m_seq | (cc == S + ii_cols[k])
        m_seq = m_seq | (cc == S + INTENT + si_cols[k])

    eyeI = (jax.lax.broadcasted_iota(jnp.int32, (INTENT, INTENT), 0) ==
            jax.lax.broadcasted_iota(jnp.int32, (INTENT, INTENT), 1))
    eyeS = (jax.lax.broadcasted_iota(jnp.int32, (SLOT, SLOT), 0) ==
            jax.lax.broadcasted_iota(jnp.int32, (SLOT, SLOT), 1))
    si_f = jnp.transpose(int_oh_T)
    ss_f = jnp.transpose(slot_oh_T)
    ri = _mm(si_f, slot_oh_T)
    rs = _mm(ss_f, int_oh_T)

    m_int = jnp.concatenate([si_f > 0.5, eyeI, ri > 0.5], axis=1)
    m_slot = jnp.concatenate([ss_f > 0.5, rs > 0.5, eyeS], axis=1)
    mask = jnp.concatenate([m_seq, m_int, m_slot], axis=0)

    hcat = jnp.concatenate([hid_ref[0], iemb_ref[...], semb_ref[...]],
                           axis=0)
    h_all = _mm(hcat, wcat_ref[...])
    a1_all = _mm(h_all, a1m_ref[...])
    a2_all_T = jnp.transpose(_mm(h_all, a2m_ref[...]))
    ones_n = jnp.ones((N, 1), F32)

    head_outs = []
    for i in range(NHEAD):
        h_aug = jnp.concatenate([h_all[:, i * GHD:(i + 1) * GHD], ones_n],
                                axis=1)
        a1c = a1_all[:, i:i + 1]
        ub = _lrelu(a1c + jnp.max(a2_all_T[i:i + 1, :]))
        epre = a1c + a2_all_T[i:i + 1, :]
        head_outs.append(_elu(_masked_softmax_mm(mask, epre, ub, h_aug, GHD)))
    x2 = jnp.concatenate(head_outs, axis=1)

    h2 = _mm(x2, wout_ref[...])
    a12 = _mm(h2, aoutc_ref[...])
    a2_2row = jnp.transpose(a12[:, 1:2])
    h2_aug = jnp.concatenate([h2, ones_n], axis=1)
    ub2 = _lrelu(a12[:S, 0:1] + jnp.max(a2_2row))
    e2pre = a12[:S, 0:1] + a2_2row
    hidden = _elu(_masked_softmax_mm(mask[:S], e2pre, ub2, h2_aug, GOD))

    hidden_ref[0] = hidden
    iout_ref[0] = _mm(hidden, iw_ref[...]) + ib_ref[...]
    sout_ref[0] = _mm(hidden, sw_ref[...]) + sb_ref[...]


@jax.jit
def _run(hiddens, intent_pro, slot_pro, intent_embedding, slot_embedding,
         gat_W, gat_a, gat_Wout, gat_aout, intent_W, intent_b, slot_W, slot_b):
    wcat = jnp.transpose(gat_W, (1, 0, 2)).reshape(H, NHEAD * GHD)
    sel = np.arange(NHEAD).repeat(GHD)
    blkmask = jnp.asarray(sel[:, None] == np.arange(NHEAD)[None, :], F32)
    a1m = blkmask * gat_a[:, :GHD].reshape(-1, 1)
    a2m = blkmask * gat_a[:, GHD:].reshape(-1, 1)
    aoutc = jnp.stack([gat_aout[:GOD], gat_aout[GOD:]], axis=1)

    iv, ii, sv, si = _sc_topk(intent_pro, slot_pro)

    fullmap = lambda b: (0, 0)
    outs = pl.pallas_call(
        _main_body,
        grid=(B,),
        in_specs=[
            pl.BlockSpec((1, S, 48), lambda b: (b, 0, 0)),
            pl.BlockSpec((1, S, 48), lambda b: (b, 0, 0)),
            pl.BlockSpec((1, S, 48), lambda b: (b, 0, 0)),
            pl.BlockSpec((1, S, 48), lambda b: (b, 0, 0)),
            pl.BlockSpec((1, S, H), lambda b: (b, 0, 0)),
            pl.BlockSpec((INTENT, H), fullmap),
            pl.BlockSpec((SLOT, H), fullmap),
            pl.BlockSpec((H, NHEAD * GHD), fullmap),
            pl.BlockSpec((NHEAD * GHD, NHEAD), fullmap),
            pl.BlockSpec((NHEAD * GHD, NHEAD), fullmap),
            pl.BlockSpec((NHEAD * GHD, GOD), fullmap),
            pl.BlockSpec((GOD, 2), fullmap),
            pl.BlockSpec((GOD, INTENT), fullmap),
            pl.BlockSpec((1, INTENT), fullmap),
            pl.BlockSpec((GOD, SLOT), fullmap),
            pl.BlockSpec((1, SLOT), fullmap),
        ],
        out_specs=(
            pl.BlockSpec((1, S, GOD), lambda b: (b, 0, 0)),
            pl.BlockSpec((1, S, INTENT), lambda b: (b, 0, 0)),
            pl.BlockSpec((1, S, SLOT), lambda b: (b, 0, 0)),
        ),
        out_shape=(
            jax.ShapeDtypeStruct((B, S, GOD), F32),
            jax.ShapeDtypeStruct((B, S, INTENT), F32),
            jax.ShapeDtypeStruct((B, S, SLOT), F32),
        ),
    )(iv, ii, sv, si, hiddens, intent_embedding, slot_embedding,
      wcat, a1m, a2m, gat_Wout, aoutc, intent_W,
      intent_b.reshape(1, INTENT), slot_W, slot_b.reshape(1, SLOT))
    return outs


def kernel(hiddens, seq_lens, intent_pro, slot_pro, intent_embedding,
           slot_embedding, gat_W, gat_a, gat_Wout, gat_aout, intent_W,
           intent_b, slot_W, slot_b):
    hidden, intent_out, slot_out = _run(
        hiddens, intent_pro, slot_pro, intent_embedding, slot_embedding,
        gat_W, gat_a, gat_Wout, gat_aout, intent_W, intent_b, slot_W, slot_b)
    return (hidden, hidden, intent_out, slot_out)


# speedup vs baseline: 1.0663x; 1.0663x over previous
import functools

import jax
import jax.numpy as jnp
import numpy as np
from jax.experimental import pallas as pl
from jax.experimental.pallas import tpu as pltpu
from jax.experimental.pallas import tpu_sc as plsc

B = 8; S = 512; H = 128; INTENT = 128; SLOT = 512
GHD = 16; GOD = 128; NHEAD = 4; TOPK = 3; WINDOW = 2; ALPHA = 0.2
N = S + INTENT + SLOT
NEG = -9e15
F32 = jnp.float32

NW = 32
RW = (B * S) // NW
WPB = S // RW


def _lrelu(x):
    return jnp.where(x >= 0, x, ALPHA * x)


def _elu(x):
    return jnp.where(x > 0, x, jnp.exp(jnp.minimum(x, 0.0)) - 1.0)


def _mm(a, b):
    return jnp.dot(a, b, preferred_element_type=F32)


def _sc_topk(intent_pro, slot_pro):
    mesh = plsc.VectorSubcoreMesh(core_axis_name="c", subcore_axis_name="s")

    @functools.partial(
        pl.kernel,
        mesh=mesh,
        out_type=(
            jax.ShapeDtypeStruct((B, S, 48), F32),
            jax.ShapeDtypeStruct((B, S, 48), jnp.int32),
            jax.ShapeDtypeStruct((B, S, 48), F32),
            jax.ShapeDtypeStruct((B, S, 48), jnp.int32),
        ),
        scratch_types=[
            pltpu.VMEM((2, 32, INTENT), F32),
            pltpu.VMEM((2, 32, SLOT), F32),
            pltpu.VMEM((32, 48), F32),
            pltpu.VMEM((32, 48), jnp.int32),
            pltpu.VMEM((32, 48), F32),
            pltpu.VMEM((32, 48), jnp.int32),
            pltpu.SemaphoreType.DMA,
            pltpu.SemaphoreType.DMA,
            pltpu.SemaphoreType.DMA,
            pltpu.SemaphoreType.DMA,
        ],
    )
    def sc_kernel(ip_hbm, sp_hbm, iv_hbm, ii_hbm, sv_hbm, si_hbm,
                  ip_v, sp_v, iv_v, ii_v, sv_v, si_v,
                  sem_ip0, sem_ip1, sem_sp0, sem_sp1):
        sem_ip = (sem_ip0, sem_ip1)
        sem_sp = (sem_sp0, sem_sp1)
        cid = jax.lax.axis_index("c")
        sid = jax.lax.axis_index("s")
        wid = sid * 2 + cid
        b = wid // WPB
        r0 = (wid % WPB) * RW
        lane = jax.lax.broadcasted_iota(jnp.int32, (16,), 0)
        negf = jnp.float32(-3.0e38)
        nblk = RW // 32

        def start_in(blk, buf):
            rb = r0 + blk * 32
            h1 = pltpu.async_copy(ip_hbm.at[b, pl.ds(rb, 32)],
                                  ip_v.at[buf], sem_ip[buf])
            h2 = pltpu.async_copy(sp_hbm.at[b, pl.ds(rb, 32)],
                                  sp_v.at[buf], sem_sp[buf])
            return (h1, h2)

        def row_top3(ref, buf, r, nchunk, vout, iout):
            z = jnp.zeros((16,), jnp.int32)
            nf = jnp.full((16,), negf, F32)
            b1, b2, b3, i1, i2, i3 = nf, nf, nf, z, z, z
            for ch in range(nchunk):
                v = ref[buf, r, pl.ds(ch * 16, 16)]
                idx = lane + (ch * 16)
                gt1 = v > b1
                gt2 = v > b2
                gt3 = v > b3
                nb1 = jnp.where(gt1, v, b1)
                ni1 = jnp.where(gt1, idx, i1)
                nb2 = jnp.where(gt1, b1, jnp.where(gt2, v, b2))
                ni2 = jnp.where(gt1, i1, jnp.where(gt2, idx, i2))
                nb3 = jnp.where(gt2, b2, jnp.where(gt3, v, b3))
                ni3 = jnp.where(gt2, i2, jnp.where(gt3, idx, i3))
                b1, i1, b2, i2, b3, i3 = nb1, ni1, nb2, ni2, nb3, ni3
            vout[r, pl.ds(0, 16)] = b1
            vout[r, pl.ds(16, 16)] = b2
            vout[r, pl.ds(32, 16)] = b3
            iout[r, pl.ds(0, 16)] = i1
            iout[r, pl.ds(16, 16)] = i2
            iout[r, pl.ds(32, 16)] = i3

        handles = {0: start_in(0, 0)}
        for blk in range(nblk):
            cur = blk % 2
            if blk + 1 < nblk:
                handles[blk + 1] = start_in(blk + 1, 1 - cur)
            h1, h2 = handles.pop(blk)
            h1.wait()
            h2.wait()

            def row_body(r, carry, _cur=cur):
                row_top3(ip_v, _cur, r, INTENT // 16, iv_v, ii_v)
                row_top3(sp_v, _cur, r, SLOT // 16, sv_v, si_v)
                return carry

            jax.lax.fori_loop(0, 32, row_body, 0)
            rb = r0 + blk * 32
            pltpu.sync_copy(iv_v, iv_hbm.at[b, pl.ds(rb, 32)])
            pltpu.sync_copy(ii_v, ii_hbm.at[b, pl.ds(rb, 32)])
            pltpu.sync_copy(sv_v, sv_hbm.at[b, pl.ds(rb, 32)])
            pltpu.sync_copy(si_v, si_hbm.at[b, pl.ds(rb, 32)])

    return sc_kernel(intent_pro, slot_pro)


def _masked_softmax_mm(mask, epre, ub, h_aug, width):
    p = jnp.where(mask, jnp.exp(_lrelu(epre) - ub), 0.0)
    pm = _mm(p, h_aug)
    return pm[:, :width] / pm[:, width:width + 1]


def _merge48(vals, idxs, width):
    cols = []
    for _ in range(TOPK):
        m = jnp.max(vals, axis=1, keepdims=True)
        cand = jnp.where(vals >= m, idxs, width)
        g = jnp.min(cand, axis=1, keepdims=True)
        cols.append(g)
        vals = jnp.where(idxs == g, -3.0e38, vals)
    return cols


def _pre_body(hid_ref, iemb_ref, semb_ref, wcat_ref, a1m_ref, a2m_ref,
              h_ref, a1_ref, a2t_ref):
    hcat = jnp.concatenate([hid_ref[0], iemb_ref[...], semb_ref[...]],
                           axis=0)
    h_all = _mm(hcat, wcat_ref[...])
    h_ref[0] = h_all
    a1_ref[0] = _mm(h_all, a1m_ref[...])
    a2t_ref[0] = jnp.transpose(_mm(h_all, a2m_ref[...]))


def _main_body(iv_ref, ii_ref, sv_ref, si_ref, h_ref, a1_ref, a2t_ref,
               wout_ref, aoutc_ref, iw_ref, ib_ref, sw_ref, sb_ref,
               hidden_ref, iout_ref, sout_ref):
    ii_cols = _merge48(iv_ref[0], ii_ref[0], INTENT)
    si_cols = _merge48(sv_ref[0], si_ref[0], SLOT)

    iotaI = jax.lax.broadcasted_iota(jnp.int32, (S, INTENT), 1)
    iotaS = jax.lax.broadcasted_iota(jnp.int32, (S, SLOT), 1)
    int_oh = (iotaI == ii_cols[0]) | (iotaI == ii_cols[1]) | (iotaI == ii_cols[2])
    slot_oh = (iotaS == si_cols[0]) | (iotaS == si_cols[1]) | (iotaS == si_cols[2])
    int_oh_T = int_oh.astype(F32)
    slot_oh_T = slot_oh.astype(F32)

    rr = jax.lax.broadcasted_iota(jnp.int32, (S, N), 0)
    cc = jax.lax.broadcasted_iota(jnp.int32, (S, N), 1)
    m_seq = (cc >= rr - WINDOW) & (cc <= rr + WINDOW)
    for k in range(TOPK):
        m_seq = m_seq | (cc == S + ii_cols[k])
        m_seq = m_seq | (cc == S + INTENT + si_cols[k])

    eyeI = (jax.lax.broadcasted_iota(jnp.int32, (INTENT, INTENT), 0) ==
            jax.lax.broadcasted_iota(jnp.int32, (INTENT, INTENT), 1))
    eyeS = (jax.lax.broadcasted_iota(jnp.int32, (SLOT, SLOT), 0) ==
            jax.lax.broadcasted_iota(jnp.int32, (SLOT, SLOT), 1))
    si_f = jnp.transpose(int_oh_T)
    ss_f = jnp.transpose(slot_oh_T)
    ri = _mm(si_f, slot_oh_T)
    rs = _mm(ss_f, int_oh_T)

    m_int = jnp.concatenate([si_f > 0.5, eyeI, ri > 0.5], axis=1)
    m_slot = jnp.concatenate([ss_f > 0.5, rs > 0.5, eyeS], axis=1)
    mask = jnp.concatenate([m_seq, m_int, m_slot], axis=0)

    h_all = h_ref[0]
    a1_all = a1_ref[0]
    a2_all_T = a2t_ref[0]
    ones_n = jnp.ones((N, 1), F32)

    head_outs = []
    for i in range(NHEAD):
        h_aug = jnp.concatenate([h_all[:, i * GHD:(i + 1) * GHD], ones_n],
                                axis=1)
        a1c = a1_all[:, i:i + 1]
        ub = _lrelu(a1c + jnp.max(a2_all_T[i:i + 1, :]))
        epre = a1c + a2_all_T[i:i + 1, :]
        head_outs.append(_elu(_masked_softmax_mm(mask, epre, ub, h_aug, GHD)))
    x2 = jnp.concatenate(head_outs, axis=1)

    h2 = _mm(x2, wout_ref[...])
    a12 = _mm(h2, aoutc_ref[...])
    a2_2row = jnp.transpose(a12[:, 1:2])
    h2_aug = jnp.concatenate([h2, ones_n], axis=1)
    ub2 = _lrelu(a12[:S, 0:1] + jnp.max(a2_2row))
    e2pre = a12[:S, 0:1] + a2_2row
    hidden = _elu(_masked_softmax_mm(mask[:S], e2pre, ub2, h2_aug, GOD))

    hidden_ref[0] = hidden
    iout_ref[0] = _mm(hidden, iw_ref[...]) + ib_ref[...]
    sout_ref[0] = _mm(hidden, sw_ref[...]) + sb_ref[...]


@jax.jit
def _run(hiddens, intent_pro, slot_pro, intent_embedding, slot_embedding,
         gat_W, gat_a, gat_Wout, gat_aout, intent_W, intent_b, slot_W, slot_b):
    wcat = jnp.transpose(gat_W, (1, 0, 2)).reshape(H, NHEAD * GHD)
    sel = np.arange(NHEAD).repeat(GHD)
    blkmask = jnp.asarray(sel[:, None] == np.arange(NHEAD)[None, :], F32)
    a1m = blkmask * gat_a[:, :GHD].reshape(-1, 1)
    a2m = blkmask * gat_a[:, GHD:].reshape(-1, 1)
    aoutc = jnp.stack([gat_aout[:GOD], gat_aout[GOD:]], axis=1)

    iv, ii, sv, si = _sc_topk(intent_pro, slot_pro)

    fullmap = lambda b: (0, 0)
    h_all, a1_all, a2t = pl.pallas_call(
        _pre_body,
        grid=(B,),
        in_specs=[
            pl.BlockSpec((1, S, H), lambda b: (b, 0, 0)),
            pl.BlockSpec((INTENT, H), fullmap),
            pl.BlockSpec((SLOT, H), fullmap),
            pl.BlockSpec((H, NHEAD * GHD), fullmap),
            pl.BlockSpec((NHEAD * GHD, NHEAD), fullmap),
            pl.BlockSpec((NHEAD * GHD, NHEAD), fullmap),
        ],
        out_specs=(
            pl.BlockSpec((1, N, NHEAD * GHD), lambda b: (b, 0, 0)),
            pl.BlockSpec((1, N, NHEAD), lambda b: (b, 0, 0)),
            pl.BlockSpec((1, NHEAD, N), lambda b: (b, 0, 0)),
        ),
        out_shape=(
            jax.ShapeDtypeStruct((B, N, NHEAD * GHD), F32),
            jax.ShapeDtypeStruct((B, N, NHEAD), F32),
            jax.ShapeDtypeStruct((B, NHEAD, N), F32),
        ),
    )(hiddens, intent_embedding, slot_embedding, wcat, a1m, a2m)

    outs = pl.pallas_call(
        _main_body,
        grid=(B,),
        in_specs=[
            pl.BlockSpec((1, S, 48), lambda b: (b, 0, 0)),
            pl.BlockSpec((1, S, 48), lambda b: (b, 0, 0)),
            pl.BlockSpec((1, S, 48), lambda b: (b, 0, 0)),
            pl.BlockSpec((1, S, 48), lambda b: (b, 0, 0)),
            pl.BlockSpec((1, N, NHEAD * GHD), lambda b: (b, 0, 0)),
            pl.BlockSpec((1, N, NHEAD), lambda b: (b, 0, 0)),
            pl.BlockSpec((1, NHEAD, N), lambda b: (b, 0, 0)),
            pl.BlockSpec((NHEAD * GHD, GOD), fullmap),
            pl.BlockSpec((GOD, 2), fullmap),
            pl.BlockSpec((GOD, INTENT), fullmap),
            pl.BlockSpec((1, INTENT), fullmap),
            pl.BlockSpec((GOD, SLOT), fullmap),
            pl.BlockSpec((1, SLOT), fullmap),
        ],
        out_specs=(
            pl.BlockSpec((1, S, GOD), lambda b: (b, 0, 0)),
            pl.BlockSpec((1, S, INTENT), lambda b: (b, 0, 0)),
            pl.BlockSpec((1, S, SLOT), lambda b: (b, 0, 0)),
        ),
        out_shape=(
            jax.ShapeDtypeStruct((B, S, GOD), F32),
            jax.ShapeDtypeStruct((B, S, INTENT), F32),
            jax.ShapeDtypeStruct((B, S, SLOT), F32),
        ),
    )(iv, ii, sv, si, h_all, a1_all, a2t, gat_Wout, aoutc, intent_W,
      intent_b.reshape(1, INTENT), slot_W, slot_b.reshape(1, SLOT))
    return outs


def kernel(hiddens, seq_lens, intent_pro, slot_pro, intent_embedding,
           slot_embedding, gat_W, gat_a, gat_Wout, gat_aout, intent_W,
           intent_b, slot_W, slot_b):
    hidden, intent_out, slot_out = _run(
        hiddens, intent_pro, slot_pro, intent_embedding, slot_embedding,
        gat_W, gat_a, gat_Wout, gat_aout, intent_W, intent_b, slot_W, slot_b)
    return (hidden, hidden, intent_out, slot_out)
